# baseline (speedup 1.0000x reference)
import functools

import jax
import jax.numpy as jnp
import numpy as np
from jax import lax
from jax.experimental import pallas as pl
from jax.experimental.pallas import tpu as pltpu
from jax.experimental.pallas import tpu_sc as plsc

_NC = 2
_NS = 16
_NW = _NC * _NS
_CH = 128
_L = 16
_HIMASK = np.uint32(0xFFFF0000)


def _pack_halves(x):
    half = x.shape[1] // 2
    hi = x[:, :half].astype(jnp.bfloat16).astype(jnp.float32)
    lo = x[:, half:].astype(jnp.bfloat16).astype(jnp.float32)
    uhi = lax.bitcast_convert_type(hi, jnp.uint32)
    ulo = lax.bitcast_convert_type(lo, jnp.uint32)
    return lax.bitcast_convert_type(uhi | (ulo >> 16), jnp.float32)


def _prep_body(lenl_ref, lenr_ref, g1_ref, g2_ref, w1a_ref, w1b_ref, b1_ref,
               idxl_ref, idxr_ref, seg_ref, a_ref, c_ref, gl_ref, gr_ref):
    a_ref[...] = _pack_halves(jnp.dot(g1_ref[...], w1a_ref[...],
                                      preferred_element_type=jnp.float32))
    c_ref[...] = _pack_halves(jnp.dot(g2_ref[...], w1b_ref[...],
                                      preferred_element_type=jnp.float32)
                              + b1_ref[...])
    seg = seg_ref[...]
    offl = jnp.zeros_like(seg)
    offr = jnp.zeros_like(seg)
    runl = jnp.int32(0)
    runr = jnp.int32(0)
    nseg = lenl_ref.shape[0]
    for s in range(nseg):
        offl = offl + jnp.where(seg == s, runl, 0)
        offr = offr + jnp.where(seg == s, runr, 0)
        runl = runl + lenl_ref[s]
        runr = runr + lenr_ref[s]
    gl_ref[...] = idxl_ref[...] + offl
    gr_ref[...] = idxr_ref[...] + offr


def _sc_body(nchunk, ppw, a_hbm, c_hbm, gl_hbm, gr_hbm, w2_hbm, b2_hbm,
             out_hbm, gl0_v, gr0_v, ra0_v, rc0_v, gl1_v, gr1_v, ra1_v, rc1_v,
             outbuf_v, w2_v, b2_v, sa0, sc0, sa1, sc1):
    cid = lax.axis_index("c")
    sid = lax.axis_index("s")
    wid = sid * _NC + cid
    base_w = wid * ppw

    pltpu.sync_copy(w2_hbm, w2_v)
    pltpu.sync_copy(b2_hbm, b2_v)
    b2vec = b2_v[...]
    w2regs = [w2_v[pl.ds(j * _L, _L)] for j in range(8)]
    lane = lax.iota(jnp.int32, _L)

    bufs = ((gl0_v, gr0_v, ra0_v, rc0_v, sa0, sc0),
            (gl1_v, gr1_v, ra1_v, rc1_v, sa1, sc1))

    def issue(buf, base):
        gl_v, gr_v, ra_v, rc_v, sem_a, sem_c = buf
        pltpu.sync_copy(gl_hbm.at[pl.ds(base, _CH)], gl_v)
        pltpu.sync_copy(gr_hbm.at[pl.ds(base, _CH)], gr_v)
        pltpu.async_copy(a_hbm.at[gl_v], ra_v, sem_a)
        pltpu.async_copy(c_hbm.at[gr_v], rc_v, sem_c)

    def drain(buf):
        gl_v, gr_v, ra_v, rc_v, sem_a, sem_c = buf
        pltpu.make_async_copy(a_hbm.at[gl_v], ra_v, sem_a).wait()
        pltpu.make_async_copy(c_hbm.at[gr_v], rc_v, sem_c).wait()

    def compute(buf, base):
        gl_v, gr_v, ra_v, rc_v, sem_a, sem_c = buf

        def group_body(g, gcarry):
            ovec = b2vec
            for i in range(_L):
                p = g * _L + i
                acc = jnp.zeros((_L,), jnp.float32)
                for j in range(4):
                    ua = lax.bitcast_convert_type(
                        ra_v[p, pl.ds(j * _L, _L)], jnp.uint32)
                    uc = lax.bitcast_convert_type(
                        rc_v[p, pl.ds(j * _L, _L)], jnp.uint32)
                    a_hi = lax.bitcast_convert_type(ua & _HIMASK, jnp.float32)
                    c_hi = lax.bitcast_convert_type(uc & _HIMASK, jnp.float32)
                    a_lo = lax.bitcast_convert_type(ua << 16, jnp.float32)
                    c_lo = lax.bitcast_convert_type(uc << 16, jnp.float32)
                    acc = (acc
                           + jnp.maximum(a_hi + c_hi, 0.0) * w2regs[j]
                           + jnp.maximum(a_lo + c_lo, 0.0) * w2regs[4 + j])
                ovec = jnp.where(lane == i, ovec + jnp.sum(acc), ovec)
            outbuf_v[pl.ds(g * _L, _L)] = ovec
            return gcarry

        lax.fori_loop(0, _CH // _L, group_body, 0)
        pltpu.sync_copy(outbuf_v, out_hbm.at[pl.ds(base, _CH)])

    half = nchunk // 2
    issue(bufs[0], base_w)

    def body2(it, carry):
        base0 = base_w + (2 * it) * _CH
        drain(bufs[0])
        issue(bufs[1], base0 + _CH)
        compute(bufs[0], base0)
        drain(bufs[1])

        @pl.when(it < half - 1)
        def _():
            issue(bufs[0], base0 + 2 * _CH)

        compute(bufs[1], base0 + _CH)
        return carry

    lax.fori_loop(0, half, body2, 0)


def kernel(graph1_x, graph2_x, idx_left, idx_right, pair_seg, g1_len, g2_len,
           W1, b1, W2, b2):
    n, d = graph1_x.shape
    ed = W1.shape[0]
    p = idx_left.shape[0]
    nseg = g1_len.shape[0]

    grid_n = 16
    row_blk = n // grid_n
    pc = 128
    pr = p // pc
    blk_r = pr // grid_n
    prep = pl.pallas_call(
        _prep_body,
        grid=(grid_n,),
        in_specs=[
            pl.BlockSpec(memory_space=pltpu.SMEM),
            pl.BlockSpec(memory_space=pltpu.SMEM),
            pl.BlockSpec((row_blk, d), lambda i: (i, 0)),
            pl.BlockSpec((row_blk, d), lambda i: (i, 0)),
            pl.BlockSpec((d, ed), lambda i: (0, 0)),
            pl.BlockSpec((d, ed), lambda i: (0, 0)),
            pl.BlockSpec((1, ed), lambda i: (0, 0)),
            pl.BlockSpec((blk_r, pc), lambda i: (i, 0)),
            pl.BlockSpec((blk_r, pc), lambda i: (i, 0)),
            pl.BlockSpec((blk_r, pc), lambda i: (i, 0)),
        ],
        out_specs=[
            pl.BlockSpec((row_blk, ed // 2), lambda i: (i, 0)),
            pl.BlockSpec((row_blk, ed // 2), lambda i: (i, 0)),
            pl.BlockSpec((blk_r, pc), lambda i: (i, 0)),
            pl.BlockSpec((blk_r, pc), lambda i: (i, 0)),
        ],
        out_shape=[
            jax.ShapeDtypeStruct((n, ed // 2), jnp.float32),
            jax.ShapeDtypeStruct((n, ed // 2), jnp.float32),
            jax.ShapeDtypeStruct((pr, pc), jnp.int32),
            jax.ShapeDtypeStruct((pr, pc), jnp.int32),
        ],
    )
    a_t, c_t, gl2, gr2 = prep(
        g1_len, g2_len, graph1_x, graph2_x, W1[:d], W1[d:], b1.reshape(1, ed),
        idx_left.reshape(pr, pc), idx_right.reshape(pr, pc),
        pair_seg.reshape(pr, pc))
    gl = gl2.reshape(p)
    gr = gr2.reshape(p)

    ppw = p // _NW
    nchunk = ppw // _CH
    mesh = plsc.VectorSubcoreMesh(core_axis_name="c", subcore_axis_name="s")
    sc_call = pl.kernel(
        functools.partial(_sc_body, nchunk, ppw),
        out_type=jax.ShapeDtypeStruct((p,), jnp.float32),
        mesh=mesh,
        compiler_params=pltpu.CompilerParams(needs_layout_passes=False,
                                             use_tc_tiling_on_sc=False),
        scratch_types=[
            pltpu.VMEM((_CH,), jnp.int32),
            pltpu.VMEM((_CH,), jnp.int32),
            pltpu.VMEM((_CH, ed // 2), jnp.float32),
            pltpu.VMEM((_CH, ed // 2), jnp.float32),
            pltpu.VMEM((_CH,), jnp.int32),
            pltpu.VMEM((_CH,), jnp.int32),
            pltpu.VMEM((_CH, ed // 2), jnp.float32),
            pltpu.VMEM((_CH, ed // 2), jnp.float32),
            pltpu.VMEM((_CH,), jnp.float32),
            pltpu.VMEM((ed,), jnp.float32),
            pltpu.VMEM((_L,), jnp.float32),
            pltpu.SemaphoreType.DMA,
            pltpu.SemaphoreType.DMA,
            pltpu.SemaphoreType.DMA,
            pltpu.SemaphoreType.DMA,
        ],
    )
    b2vec = jnp.full((_L,), b2[0], dtype=jnp.float32)
    out = sc_call(a_t, c_t, gl, gr, W2.reshape(ed), b2vec)
    return out.reshape(p, 1)


# speedup vs baseline: 1.0483x; 1.0483x over previous
import functools

import jax
import jax.numpy as jnp
import numpy as np
from jax import lax
from jax.experimental import pallas as pl
from jax.experimental.pallas import tpu as pltpu
from jax.experimental.pallas import tpu_sc as plsc

_NC = 2
_NS = 16
_NW = _NC * _NS
_CH = 128
_L = 16
_HIMASK = np.uint32(0xFFFF0000)


def _prep_body(lenl_ref, lenr_ref, g1_ref, g2_ref, w1a_ref, w1b_ref, b1_ref,
               idxl_ref, idxr_ref, seg_ref, a_ref, c_ref, gl_ref, gr_ref):
    a_ref[...] = jnp.dot(g1_ref[...], w1a_ref[...],
                         preferred_element_type=jnp.float32)
    c_ref[...] = jnp.dot(g2_ref[...], w1b_ref[...],
                         preferred_element_type=jnp.float32) + b1_ref[...]
    seg = seg_ref[...]
    offl = jnp.zeros_like(seg)
    offr = jnp.zeros_like(seg)
    runl = jnp.int32(0)
    runr = jnp.int32(0)
    nseg = lenl_ref.shape[0]
    for s in range(nseg):
        offl = offl + jnp.where(seg == s, runl, 0)
        offr = offr + jnp.where(seg == s, runr, 0)
        runl = runl + lenl_ref[s]
        runr = runr + lenr_ref[s]
    gl_ref[...] = idxl_ref[...] + offl
    gr_ref[...] = idxr_ref[...] + offr


_DEPTH = 3


def _sc_body(nchunk, ppw, a_hbm, c_hbm, gl_hbm, gr_hbm, w2_hbm, b2_hbm,
             out_hbm, *scratch):
    nb = 4 * _DEPTH
    outbuf_v, w2_v, b2_v = scratch[nb:nb + 3]
    sems = scratch[nb + 3:]
    bufs = tuple(tuple(scratch[4 * k:4 * k + 4]) + tuple(sems[2 * k:2 * k + 2])
                 for k in range(_DEPTH))

    cid = lax.axis_index("c")
    sid = lax.axis_index("s")
    wid = sid * _NC + cid
    base_w = wid * ppw

    pltpu.sync_copy(w2_hbm, w2_v)
    pltpu.sync_copy(b2_hbm, b2_v)
    b2vec = b2_v[...]
    w2regs = [w2_v[pl.ds(j * _L, _L)] for j in range(8)]
    lane = lax.iota(jnp.int32, _L)

    def issue(buf, base):
        gl_v, gr_v, ra_v, rc_v, sem_a, sem_c = buf
        pltpu.sync_copy(gl_hbm.at[pl.ds(base, _CH)], gl_v)
        pltpu.sync_copy(gr_hbm.at[pl.ds(base, _CH)], gr_v)
        pltpu.async_copy(a_hbm.at[gl_v], ra_v, sem_a)
        pltpu.async_copy(c_hbm.at[gr_v], rc_v, sem_c)

    def drain(buf):
        gl_v, gr_v, ra_v, rc_v, sem_a, sem_c = buf
        pltpu.make_async_copy(a_hbm.at[gl_v], ra_v, sem_a).wait()
        pltpu.make_async_copy(c_hbm.at[gr_v], rc_v, sem_c).wait()

    def compute(buf, base):
        gl_v, gr_v, ra_v, rc_v, sem_a, sem_c = buf

        def group_body(g, gcarry):
            ovec = b2vec
            for i in range(_L):
                p = g * _L + i
                acc = jnp.zeros((_L,), jnp.float32)
                for j in range(8):
                    va = ra_v[p, pl.ds(j * _L, _L)]
                    vc = rc_v[p, pl.ds(j * _L, _L)]
                    acc = acc + jnp.maximum(va + vc, 0.0) * w2regs[j]
                ovec = jnp.where(lane == i, ovec + jnp.sum(acc), ovec)
            outbuf_v[pl.ds(g * _L, _L)] = ovec
            return gcarry

        lax.fori_loop(0, _CH // _L, group_body, 0)
        pltpu.sync_copy(outbuf_v, out_hbm.at[pl.ds(base, _CH)])

    for k in range(_DEPTH - 1):
        issue(bufs[k], base_w + k * _CH)

    rounds = nchunk // _DEPTH
    tail = nchunk % _DEPTH

    def body(it, carry):
        base0 = base_w + (_DEPTH * it) * _CH
        for k in range(_DEPTH):
            ch_next = _DEPTH * it + k + _DEPTH - 1
            drain(bufs[k])

            @pl.when(ch_next < nchunk)
            def _(k=k, ch_next=ch_next):
                issue(bufs[(k + _DEPTH - 1) % _DEPTH],
                      base_w + ch_next * _CH)

            compute(bufs[k], base0 + k * _CH)
        return carry

    lax.fori_loop(0, rounds, body, 0)
    for k in range(tail):
        ch = rounds * _DEPTH + k
        drain(bufs[k])
        compute(bufs[k], base_w + ch * _CH)


def kernel(graph1_x, graph2_x, idx_left, idx_right, pair_seg, g1_len, g2_len,
           W1, b1, W2, b2):
    n, d = graph1_x.shape
    ed = W1.shape[0]
    p = idx_left.shape[0]
    nseg = g1_len.shape[0]

    grid_n = 16
    row_blk = n // grid_n
    pc = 128
    pr = p // pc
    blk_r = pr // grid_n
    prep = pl.pallas_call(
        _prep_body,
        grid=(grid_n,),
        in_specs=[
            pl.BlockSpec(memory_space=pltpu.SMEM),
            pl.BlockSpec(memory_space=pltpu.SMEM),
            pl.BlockSpec((row_blk, d), lambda i: (i, 0)),
            pl.BlockSpec((row_blk, d), lambda i: (i, 0)),
            pl.BlockSpec((d, ed), lambda i: (0, 0)),
            pl.BlockSpec((d, ed), lambda i: (0, 0)),
            pl.BlockSpec((1, ed), lambda i: (0, 0)),
            pl.BlockSpec((blk_r, pc), lambda i: (i, 0)),
            pl.BlockSpec((blk_r, pc), lambda i: (i, 0)),
            pl.BlockSpec((blk_r, pc), lambda i: (i, 0)),
        ],
        out_specs=[
            pl.BlockSpec((row_blk, ed), lambda i: (i, 0)),
            pl.BlockSpec((row_blk, ed), lambda i: (i, 0)),
            pl.BlockSpec((blk_r, pc), lambda i: (i, 0)),
            pl.BlockSpec((blk_r, pc), lambda i: (i, 0)),
        ],
        out_shape=[
            jax.ShapeDtypeStruct((n, ed), jnp.float32),
            jax.ShapeDtypeStruct((n, ed), jnp.float32),
            jax.ShapeDtypeStruct((pr, pc), jnp.int32),
            jax.ShapeDtypeStruct((pr, pc), jnp.int32),
        ],
    )
    a_t, c_t, gl2, gr2 = prep(
        g1_len, g2_len, graph1_x, graph2_x, W1[:d], W1[d:], b1.reshape(1, ed),
        idx_left.reshape(pr, pc), idx_right.reshape(pr, pc),
        pair_seg.reshape(pr, pc))
    gl = gl2.reshape(p)
    gr = gr2.reshape(p)

    ppw = p // _NW
    nchunk = ppw // _CH
    mesh = plsc.VectorSubcoreMesh(core_axis_name="c", subcore_axis_name="s")
    sc_call = pl.kernel(
        functools.partial(_sc_body, nchunk, ppw),
        out_type=jax.ShapeDtypeStruct((p,), jnp.float32),
        mesh=mesh,
        compiler_params=pltpu.CompilerParams(needs_layout_passes=False),
        scratch_types=(
            [pltpu.VMEM((_CH,), jnp.int32),
             pltpu.VMEM((_CH,), jnp.int32),
             pltpu.VMEM((_CH, ed), jnp.float32),
             pltpu.VMEM((_CH, ed), jnp.float32)] * _DEPTH
            + [pltpu.VMEM((_CH,), jnp.float32),
               pltpu.VMEM((ed,), jnp.float32),
               pltpu.VMEM((_L,), jnp.float32)]
            + [pltpu.SemaphoreType.DMA] * (2 * _DEPTH)
        ),
    )
    b2vec = jnp.full((_L,), b2[0], dtype=jnp.float32)
    out = sc_call(a_t, c_t, gl, gr, W2.reshape(ed), b2vec)
    return out.reshape(p, 1)


# speedup vs baseline: 1.2449x; 1.1875x over previous
import functools

import jax
import jax.numpy as jnp
import numpy as np
from jax import lax
from jax.experimental import pallas as pl
from jax.experimental.pallas import tpu as pltpu
from jax.experimental.pallas import tpu_sc as plsc

_NC = 2
_NS = 16
_NW = _NC * _NS
_CH = 128
_L = 16
_HIMASK = np.uint32(0xFFFF0000)


def _prep_body(lenl_ref, lenr_ref, g1t_ref, g2t_ref, w1a_ref, w1b_ref, b1_ref,
               idxl_ref, idxr_ref, seg_ref, a_ref, c_ref, gl_ref, gr_ref):
    a_ref[...] = lax.dot_general(
        g1t_ref[...], w1a_ref[...], (((0,), (0,)), ((), ())),
        preferred_element_type=jnp.float32)
    c_ref[...] = lax.dot_general(
        g2t_ref[...], w1b_ref[...], (((0,), (0,)), ((), ())),
        preferred_element_type=jnp.float32) + b1_ref[...]
    seg = seg_ref[...]
    offl = jnp.zeros_like(seg)
    offr = jnp.zeros_like(seg)
    runl = jnp.int32(0)
    runr = jnp.int32(0)
    nseg = lenl_ref.shape[0]
    for s in range(nseg):
        offl = offl + jnp.where(seg == s, runl, 0)
        offr = offr + jnp.where(seg == s, runr, 0)
        runl = runl + lenl_ref[s]
        runr = runr + lenr_ref[s]
    gl_ref[...] = idxl_ref[...] + offl
    gr_ref[...] = idxr_ref[...] + offr


_DEPTH = 3


def _sc_body(nchunk, ppw, a_hbm, c_hbm, gl_hbm, gr_hbm, w2_hbm, b2_hbm,
             out_hbm, *scratch):
    nb = 4 * _DEPTH
    outbuf_v, w2_v, b2_v = scratch[nb:nb + 3]
    sems = scratch[nb + 3:]
    bufs = tuple(tuple(scratch[4 * k:4 * k + 4]) + tuple(sems[2 * k:2 * k + 2])
                 for k in range(_DEPTH))

    cid = lax.axis_index("c")
    sid = lax.axis_index("s")
    wid = sid * _NC + cid
    base_w = wid * ppw

    pltpu.sync_copy(w2_hbm, w2_v)
    pltpu.sync_copy(b2_hbm, b2_v)
    b2vec = b2_v[...]
    w2regs = [w2_v[pl.ds(j * _L, _L)] for j in range(8)]
    lane = lax.iota(jnp.int32, _L)

    def issue(buf, base):
        gl_v, gr_v, ra_v, rc_v, sem_a, sem_c = buf
        pltpu.sync_copy(gl_hbm.at[pl.ds(base, _CH)], gl_v)
        pltpu.sync_copy(gr_hbm.at[pl.ds(base, _CH)], gr_v)
        pltpu.async_copy(a_hbm.at[gl_v], ra_v, sem_a)
        pltpu.async_copy(c_hbm.at[gr_v], rc_v, sem_c)

    def drain(buf):
        gl_v, gr_v, ra_v, rc_v, sem_a, sem_c = buf
        pltpu.make_async_copy(a_hbm.at[gl_v], ra_v, sem_a).wait()
        pltpu.make_async_copy(c_hbm.at[gr_v], rc_v, sem_c).wait()

    def compute(buf, base):
        gl_v, gr_v, ra_v, rc_v, sem_a, sem_c = buf

        def group_body(g, gcarry):
            ovec = b2vec
            for i in range(_L):
                p = g * _L + i
                acc = jnp.zeros((_L,), jnp.float32)
                for j in range(8):
                    va = ra_v[p, pl.ds(j * _L, _L)]
                    vc = rc_v[p, pl.ds(j * _L, _L)]
                    acc = acc + jnp.maximum(va + vc, 0.0) * w2regs[j]
                ovec = jnp.where(lane == i, ovec + jnp.sum(acc), ovec)
            outbuf_v[pl.ds(g * _L, _L)] = ovec
            return gcarry

        lax.fori_loop(0, _CH // _L, group_body, 0)
        pltpu.sync_copy(outbuf_v, out_hbm.at[pl.ds(base, _CH)])

    for k in range(_DEPTH - 1):
        issue(bufs[k], base_w + k * _CH)

    rounds = nchunk // _DEPTH
    tail = nchunk % _DEPTH

    def body(it, carry):
        base0 = base_w + (_DEPTH * it) * _CH
        for k in range(_DEPTH):
            ch_next = _DEPTH * it + k + _DEPTH - 1
            drain(bufs[k])

            @pl.when(ch_next < nchunk)
            def _(k=k, ch_next=ch_next):
                issue(bufs[(k + _DEPTH - 1) % _DEPTH],
                      base_w + ch_next * _CH)

            compute(bufs[k], base0 + k * _CH)
        return carry

    lax.fori_loop(0, rounds, body, 0)
    for k in range(tail):
        ch = rounds * _DEPTH + k
        drain(bufs[k])
        compute(bufs[k], base_w + ch * _CH)


def kernel(graph1_x, graph2_x, idx_left, idx_right, pair_seg, g1_len, g2_len,
           W1, b1, W2, b2):
    n, d = graph1_x.shape
    ed = W1.shape[0]
    p = idx_left.shape[0]
    nseg = g1_len.shape[0]

    grid_n = 16
    row_blk = n // grid_n
    pc = 128
    pr = p // pc
    blk_r = pr // grid_n
    prep = pl.pallas_call(
        _prep_body,
        grid=(grid_n,),
        in_specs=[
            pl.BlockSpec(memory_space=pltpu.SMEM),
            pl.BlockSpec(memory_space=pltpu.SMEM),
            pl.BlockSpec((d, row_blk), lambda i: (0, i)),
            pl.BlockSpec((d, row_blk), lambda i: (0, i)),
            pl.BlockSpec((d, ed), lambda i: (0, 0)),
            pl.BlockSpec((d, ed), lambda i: (0, 0)),
            pl.BlockSpec((1, ed), lambda i: (0, 0)),
            pl.BlockSpec((blk_r, pc), lambda i: (i, 0)),
            pl.BlockSpec((blk_r, pc), lambda i: (i, 0)),
            pl.BlockSpec((blk_r, pc), lambda i: (i, 0)),
        ],
        out_specs=[
            pl.BlockSpec((row_blk, ed), lambda i: (i, 0)),
            pl.BlockSpec((row_blk, ed), lambda i: (i, 0)),
            pl.BlockSpec((blk_r, pc), lambda i: (i, 0)),
            pl.BlockSpec((blk_r, pc), lambda i: (i, 0)),
        ],
        out_shape=[
            jax.ShapeDtypeStruct((n, ed), jnp.float32),
            jax.ShapeDtypeStruct((n, ed), jnp.float32),
            jax.ShapeDtypeStruct((pr, pc), jnp.int32),
            jax.ShapeDtypeStruct((pr, pc), jnp.int32),
        ],
    )
    a_t, c_t, gl2, gr2 = prep(
        g1_len, g2_len, graph1_x.T, graph2_x.T, W1[:d], W1[d:],
        b1.reshape(1, ed), idx_left.reshape(pr, pc),
        idx_right.reshape(pr, pc), pair_seg.reshape(pr, pc))
    gl = gl2.reshape(p)
    gr = gr2.reshape(p)

    ppw = p // _NW
    nchunk = ppw // _CH
    mesh = plsc.VectorSubcoreMesh(core_axis_name="c", subcore_axis_name="s")
    sc_call = pl.kernel(
        functools.partial(_sc_body, nchunk, ppw),
        out_type=jax.ShapeDtypeStruct((p,), jnp.float32),
        mesh=mesh,
        compiler_params=pltpu.CompilerParams(needs_layout_passes=False),
        scratch_types=(
            [pltpu.VMEM((_CH,), jnp.int32),
             pltpu.VMEM((_CH,), jnp.int32),
             pltpu.VMEM((_CH, ed), jnp.float32),
             pltpu.VMEM((_CH, ed), jnp.float32)] * _DEPTH
            + [pltpu.VMEM((_CH,), jnp.float32),
               pltpu.VMEM((ed,), jnp.float32),
               pltpu.VMEM((_L,), jnp.float32)]
            + [pltpu.SemaphoreType.DMA] * (2 * _DEPTH)
        ),
    )
    b2vec = jnp.full((_L,), b2[0], dtype=jnp.float32)
    out = sc_call(a_t, c_t, gl, gr, W2.reshape(ed), b2vec)
    return out.reshape(p, 1)


# speedup vs baseline: 1.4279x; 1.1470x over previous
import functools

import jax
import jax.numpy as jnp
import numpy as np
from jax import lax
from jax.experimental import pallas as pl
from jax.experimental.pallas import tpu as pltpu
from jax.experimental.pallas import tpu_sc as plsc

_NC = 2
_NS = 16
_NW = _NC * _NS
_CH = 128
_L = 16
_HIMASK = np.uint32(0xFFFF0000)


def _pack_halves(x):
    half = x.shape[1] // 2
    hi = x[:, :half].astype(jnp.bfloat16).astype(jnp.float32)
    lo = x[:, half:].astype(jnp.bfloat16).astype(jnp.float32)
    uhi = lax.bitcast_convert_type(hi, jnp.uint32)
    ulo = lax.bitcast_convert_type(lo, jnp.uint32)
    packed = lax.bitcast_convert_type(uhi | (ulo >> 16), jnp.float32)
    return jnp.concatenate([packed, jnp.zeros_like(packed)], axis=1)


def _prep_body(lenl_ref, lenr_ref, g1t_ref, g2t_ref, w1a_ref, w1b_ref, b1_ref,
               idxl_ref, idxr_ref, seg_ref, a_ref, c_ref, gl_ref, gr_ref):
    a_ref[...] = _pack_halves(lax.dot_general(
        g1t_ref[...], w1a_ref[...], (((0,), (0,)), ((), ())),
        preferred_element_type=jnp.float32))
    c_ref[...] = _pack_halves(lax.dot_general(
        g2t_ref[...], w1b_ref[...], (((0,), (0,)), ((), ())),
        preferred_element_type=jnp.float32) + b1_ref[...])
    seg = seg_ref[...]
    offl = jnp.zeros_like(seg)
    offr = jnp.zeros_like(seg)
    runl = jnp.int32(0)
    runr = jnp.int32(0)
    nseg = lenl_ref.shape[0]
    for s in range(nseg):
        offl = offl + jnp.where(seg == s, runl, 0)
        offr = offr + jnp.where(seg == s, runr, 0)
        runl = runl + lenl_ref[s]
        runr = runr + lenr_ref[s]
    gl_ref[...] = (idxl_ref[...] + offl) * 2
    gr_ref[...] = (idxr_ref[...] + offr) * 2


_DEPTH = 4


def _sc_body(nchunk, ppw, a_hbm, c_hbm, gl_hbm, gr_hbm, w2_hbm, b2_hbm,
             out_hbm, *scratch):
    nb = 4 * _DEPTH
    outbuf_v, w2_v, b2_v = scratch[nb:nb + 3]
    sems = scratch[nb + 3:]
    bufs = tuple(tuple(scratch[4 * k:4 * k + 4]) + tuple(sems[2 * k:2 * k + 2])
                 for k in range(_DEPTH))

    cid = lax.axis_index("c")
    sid = lax.axis_index("s")
    wid = sid * _NC + cid
    base_w = wid * ppw

    pltpu.sync_copy(w2_hbm, w2_v)
    pltpu.sync_copy(b2_hbm, b2_v)
    b2vec = b2_v[...]
    w2regs = [w2_v[pl.ds(j * _L, _L)] for j in range(8)]
    lane = lax.iota(jnp.int32, _L)

    def issue(buf, base):
        gl_v, gr_v, ra_v, rc_v, sem_a, sem_c = buf
        pltpu.sync_copy(gl_hbm.at[pl.ds(base, _CH)], gl_v)
        pltpu.sync_copy(gr_hbm.at[pl.ds(base, _CH)], gr_v)
        pltpu.async_copy(a_hbm.at[gl_v], ra_v, sem_a)
        pltpu.async_copy(c_hbm.at[gr_v], rc_v, sem_c)

    def drain(buf):
        gl_v, gr_v, ra_v, rc_v, sem_a, sem_c = buf
        pltpu.make_async_copy(a_hbm.at[gl_v], ra_v, sem_a).wait()
        pltpu.make_async_copy(c_hbm.at[gr_v], rc_v, sem_c).wait()

    def compute(buf, base):
        gl_v, gr_v, ra_v, rc_v, sem_a, sem_c = buf

        def group_body(g, gcarry):
            ovec = b2vec
            for i in range(_L):
                p = g * _L + i
                acc = jnp.zeros((_L,), jnp.float32)
                for j in range(4):
                    ua = lax.bitcast_convert_type(
                        ra_v[p, pl.ds(j * _L, _L)], jnp.uint32)
                    uc = lax.bitcast_convert_type(
                        rc_v[p, pl.ds(j * _L, _L)], jnp.uint32)
                    a_hi = lax.bitcast_convert_type(ua & _HIMASK, jnp.float32)
                    c_hi = lax.bitcast_convert_type(uc & _HIMASK, jnp.float32)
                    a_lo = lax.bitcast_convert_type(ua << 16, jnp.float32)
                    c_lo = lax.bitcast_convert_type(uc << 16, jnp.float32)
                    acc = (acc
                           + jnp.maximum(a_hi + c_hi, 0.0) * w2regs[j]
                           + jnp.maximum(a_lo + c_lo, 0.0) * w2regs[4 + j])
                ovec = jnp.where(lane == i, ovec + jnp.sum(acc), ovec)
            outbuf_v[pl.ds(g * _L, _L)] = ovec
            return gcarry

        lax.fori_loop(0, _CH // _L, group_body, 0)
        pltpu.sync_copy(outbuf_v, out_hbm.at[pl.ds(base, _CH)])

    for k in range(_DEPTH - 1):
        issue(bufs[k], base_w + k * _CH)

    rounds = nchunk // _DEPTH
    tail = nchunk % _DEPTH

    def body(it, carry):
        base0 = base_w + (_DEPTH * it) * _CH
        for k in range(_DEPTH):
            ch_next = _DEPTH * it + k + _DEPTH - 1
            drain(bufs[k])

            @pl.when(ch_next < nchunk)
            def _(k=k, ch_next=ch_next):
                issue(bufs[(k + _DEPTH - 1) % _DEPTH],
                      base_w + ch_next * _CH)

            compute(bufs[k], base0 + k * _CH)
        return carry

    lax.fori_loop(0, rounds, body, 0)
    for k in range(tail):
        ch = rounds * _DEPTH + k
        drain(bufs[k])
        compute(bufs[k], base_w + ch * _CH)


def kernel(graph1_x, graph2_x, idx_left, idx_right, pair_seg, g1_len, g2_len,
           W1, b1, W2, b2):
    n, d = graph1_x.shape
    ed = W1.shape[0]
    p = idx_left.shape[0]
    nseg = g1_len.shape[0]

    grid_n = 16
    row_blk = n // grid_n
    pc = 128
    pr = p // pc
    blk_r = pr // grid_n
    prep = pl.pallas_call(
        _prep_body,
        grid=(grid_n,),
        in_specs=[
            pl.BlockSpec(memory_space=pltpu.SMEM),
            pl.BlockSpec(memory_space=pltpu.SMEM),
            pl.BlockSpec((d, row_blk), lambda i: (0, i)),
            pl.BlockSpec((d, row_blk), lambda i: (0, i)),
            pl.BlockSpec((d, ed), lambda i: (0, 0)),
            pl.BlockSpec((d, ed), lambda i: (0, 0)),
            pl.BlockSpec((1, ed), lambda i: (0, 0)),
            pl.BlockSpec((blk_r, pc), lambda i: (i, 0)),
            pl.BlockSpec((blk_r, pc), lambda i: (i, 0)),
            pl.BlockSpec((blk_r, pc), lambda i: (i, 0)),
        ],
        out_specs=[
            pl.BlockSpec((row_blk, ed), lambda i: (i, 0)),
            pl.BlockSpec((row_blk, ed), lambda i: (i, 0)),
            pl.BlockSpec((blk_r, pc), lambda i: (i, 0)),
            pl.BlockSpec((blk_r, pc), lambda i: (i, 0)),
        ],
        out_shape=[
            jax.ShapeDtypeStruct((n, ed), jnp.float32),
            jax.ShapeDtypeStruct((n, ed), jnp.float32),
            jax.ShapeDtypeStruct((pr, pc), jnp.int32),
            jax.ShapeDtypeStruct((pr, pc), jnp.int32),
        ],
    )
    a_t, c_t, gl2, gr2 = prep(
        g1_len, g2_len, graph1_x.T, graph2_x.T, W1[:d], W1[d:],
        b1.reshape(1, ed), idx_left.reshape(pr, pc),
        idx_right.reshape(pr, pc), pair_seg.reshape(pr, pc))
    gl = gl2.reshape(p)
    gr = gr2.reshape(p)

    ppw = p // _NW
    nchunk = ppw // _CH
    mesh = plsc.VectorSubcoreMesh(core_axis_name="c", subcore_axis_name="s")
    sc_call = pl.kernel(
        functools.partial(_sc_body, nchunk, ppw),
        out_type=jax.ShapeDtypeStruct((p,), jnp.float32),
        mesh=mesh,
        compiler_params=pltpu.CompilerParams(needs_layout_passes=False,
                                             use_tc_tiling_on_sc=False),
        scratch_types=(
            [pltpu.VMEM((_CH,), jnp.int32),
             pltpu.VMEM((_CH,), jnp.int32),
             pltpu.VMEM((_CH, ed // 2), jnp.float32),
             pltpu.VMEM((_CH, ed // 2), jnp.float32)] * _DEPTH
            + [pltpu.VMEM((_CH,), jnp.float32),
               pltpu.VMEM((ed,), jnp.float32),
               pltpu.VMEM((_L,), jnp.float32)]
            + [pltpu.SemaphoreType.DMA] * (2 * _DEPTH)
        ),
    )
    b2vec = jnp.full((_L,), b2[0], dtype=jnp.float32)
    out = sc_call(a_t.reshape(2 * n, ed // 2), c_t.reshape(2 * n, ed // 2),
                  gl, gr, W2.reshape(ed), b2vec)
    return out.reshape(p, 1)


# speedup vs baseline: 1.4293x; 1.0009x over previous
import functools

import jax
import jax.numpy as jnp
import numpy as np
from jax import lax
from jax.experimental import pallas as pl
from jax.experimental.pallas import tpu as pltpu
from jax.experimental.pallas import tpu_sc as plsc

_NC = 2
_NS = 16
_NW = _NC * _NS
_CH = 128
_L = 16
_HIMASK = np.uint32(0xFFFF0000)


def _pack_halves(x):
    half = x.shape[1] // 2
    hi = x[:, :half].astype(jnp.bfloat16).astype(jnp.float32)
    lo = x[:, half:].astype(jnp.bfloat16).astype(jnp.float32)
    uhi = lax.bitcast_convert_type(hi, jnp.uint32)
    ulo = lax.bitcast_convert_type(lo, jnp.uint32)
    return lax.bitcast_convert_type(uhi | (ulo >> 16), jnp.float32)


def _prep_body(lenl_ref, lenr_ref, g1t_ref, g2t_ref, w1a_ref, w1b_ref, b1_ref,
               idxl_ref, idxr_ref, seg_ref, t_ref, gl_ref, gr_ref):
    pa = _pack_halves(lax.dot_general(
        g1t_ref[...], w1a_ref[...], (((0,), (0,)), ((), ())),
        preferred_element_type=jnp.float32))
    pc = _pack_halves(lax.dot_general(
        g2t_ref[...], w1b_ref[...], (((0,), (0,)), ((), ())),
        preferred_element_type=jnp.float32) + b1_ref[...])
    t_ref[...] = jnp.concatenate([pa, pc], axis=1)
    seg = seg_ref[...]
    offl = jnp.zeros_like(seg)
    offr = jnp.zeros_like(seg)
    runl = jnp.int32(0)
    runr = jnp.int32(0)
    nseg = lenl_ref.shape[0]
    for s in range(nseg):
        offl = offl + jnp.where(seg == s, runl, 0)
        offr = offr + jnp.where(seg == s, runr, 0)
        runl = runl + lenl_ref[s]
        runr = runr + lenr_ref[s]
    gl_ref[...] = (idxl_ref[...] + offl) * 2
    gr_ref[...] = (idxr_ref[...] + offr) * 2 + 1


_DEPTH = 4


def _sc_body(nchunk, ppw, t_hbm, gl_hbm, gr_hbm, w2_hbm, b2_hbm,
             out_hbm, *scratch):
    nb = 4 * _DEPTH
    outbuf_v, w2_v, b2_v = scratch[nb:nb + 3]
    sems = scratch[nb + 3:]
    bufs = tuple(tuple(scratch[4 * k:4 * k + 4]) + tuple(sems[2 * k:2 * k + 2])
                 for k in range(_DEPTH))

    cid = lax.axis_index("c")
    sid = lax.axis_index("s")
    wid = sid * _NC + cid
    base_w = wid * ppw

    pltpu.sync_copy(w2_hbm, w2_v)
    pltpu.sync_copy(b2_hbm, b2_v)
    b2vec = b2_v[...]
    w2regs = [w2_v[pl.ds(j * _L, _L)] for j in range(8)]
    lane = lax.iota(jnp.int32, _L)

    def issue(buf, base):
        gl_v, gr_v, ra_v, rc_v, sem_a, sem_c = buf
        pltpu.sync_copy(gl_hbm.at[pl.ds(base, _CH)], gl_v)
        pltpu.sync_copy(gr_hbm.at[pl.ds(base, _CH)], gr_v)
        pltpu.async_copy(t_hbm.at[gl_v], ra_v, sem_a)
        pltpu.async_copy(t_hbm.at[gr_v], rc_v, sem_c)

    def drain(buf):
        gl_v, gr_v, ra_v, rc_v, sem_a, sem_c = buf
        pltpu.make_async_copy(t_hbm.at[gl_v], ra_v, sem_a).wait()
        pltpu.make_async_copy(t_hbm.at[gr_v], rc_v, sem_c).wait()

    def compute(buf, base):
        gl_v, gr_v, ra_v, rc_v, sem_a, sem_c = buf

        def group_body(g, gcarry):
            ovec = b2vec
            for i in range(_L):
                p = g * _L + i
                acc = jnp.zeros((_L,), jnp.float32)
                for j in range(4):
                    ua = lax.bitcast_convert_type(
                        ra_v[p, pl.ds(j * _L, _L)], jnp.uint32)
                    uc = lax.bitcast_convert_type(
                        rc_v[p, pl.ds(j * _L, _L)], jnp.uint32)
                    a_hi = lax.bitcast_convert_type(ua & _HIMASK, jnp.float32)
                    c_hi = lax.bitcast_convert_type(uc & _HIMASK, jnp.float32)
                    a_lo = lax.bitcast_convert_type(ua << 16, jnp.float32)
                    c_lo = lax.bitcast_convert_type(uc << 16, jnp.float32)
                    acc = (acc
                           + jnp.maximum(a_hi + c_hi, 0.0) * w2regs[j]
                           + jnp.maximum(a_lo + c_lo, 0.0) * w2regs[4 + j])
                ovec = jnp.where(lane == i, ovec + jnp.sum(acc), ovec)
            outbuf_v[pl.ds(g * _L, _L)] = ovec
            return gcarry

        lax.fori_loop(0, _CH // _L, group_body, 0)
        pltpu.sync_copy(outbuf_v, out_hbm.at[pl.ds(base, _CH)])

    for k in range(_DEPTH - 1):
        issue(bufs[k], base_w + k * _CH)

    rounds = nchunk // _DEPTH
    tail = nchunk % _DEPTH

    def body(it, carry):
        base0 = base_w + (_DEPTH * it) * _CH
        for k in range(_DEPTH):
            ch_next = _DEPTH * it + k + _DEPTH - 1
            drain(bufs[k])

            @pl.when(ch_next < nchunk)
            def _(k=k, ch_next=ch_next):
                issue(bufs[(k + _DEPTH - 1) % _DEPTH],
                      base_w + ch_next * _CH)

            compute(bufs[k], base0 + k * _CH)
        return carry

    lax.fori_loop(0, rounds, body, 0)
    for k in range(tail):
        ch = rounds * _DEPTH + k
        drain(bufs[k])
        compute(bufs[k], base_w + ch * _CH)


def kernel(graph1_x, graph2_x, idx_left, idx_right, pair_seg, g1_len, g2_len,
           W1, b1, W2, b2):
    n, d = graph1_x.shape
    ed = W1.shape[0]
    p = idx_left.shape[0]
    nseg = g1_len.shape[0]

    grid_n = 16
    row_blk = n // grid_n
    pc = 128
    pr = p // pc
    blk_r = pr // grid_n
    prep = pl.pallas_call(
        _prep_body,
        grid=(grid_n,),
        in_specs=[
            pl.BlockSpec(memory_space=pltpu.SMEM),
            pl.BlockSpec(memory_space=pltpu.SMEM),
            pl.BlockSpec((d, row_blk), lambda i: (0, i)),
            pl.BlockSpec((d, row_blk), lambda i: (0, i)),
            pl.BlockSpec((d, ed), lambda i: (0, 0)),
            pl.BlockSpec((d, ed), lambda i: (0, 0)),
            pl.BlockSpec((1, ed), lambda i: (0, 0)),
            pl.BlockSpec((blk_r, pc), lambda i: (i, 0)),
            pl.BlockSpec((blk_r, pc), lambda i: (i, 0)),
            pl.BlockSpec((blk_r, pc), lambda i: (i, 0)),
        ],
        out_specs=[
            pl.BlockSpec((row_blk, ed), lambda i: (i, 0)),
            pl.BlockSpec((blk_r, pc), lambda i: (i, 0)),
            pl.BlockSpec((blk_r, pc), lambda i: (i, 0)),
        ],
        out_shape=[
            jax.ShapeDtypeStruct((n, ed), jnp.float32),
            jax.ShapeDtypeStruct((pr, pc), jnp.int32),
            jax.ShapeDtypeStruct((pr, pc), jnp.int32),
        ],
    )
    t_t, gl2, gr2 = prep(
        g1_len, g2_len, graph1_x.T, graph2_x.T, W1[:d], W1[d:],
        b1.reshape(1, ed), idx_left.reshape(pr, pc),
        idx_right.reshape(pr, pc), pair_seg.reshape(pr, pc))
    gl = gl2.reshape(p)
    gr = gr2.reshape(p)

    ppw = p // _NW
    nchunk = ppw // _CH
    mesh = plsc.VectorSubcoreMesh(core_axis_name="c", subcore_axis_name="s")
    sc_call = pl.kernel(
        functools.partial(_sc_body, nchunk, ppw),
        out_type=jax.ShapeDtypeStruct((p,), jnp.float32),
        mesh=mesh,
        compiler_params=pltpu.CompilerParams(needs_layout_passes=False,
                                             use_tc_tiling_on_sc=False),
        scratch_types=(
            [pltpu.VMEM((_CH,), jnp.int32),
             pltpu.VMEM((_CH,), jnp.int32),
             pltpu.VMEM((_CH, ed // 2), jnp.float32),
             pltpu.VMEM((_CH, ed // 2), jnp.float32)] * _DEPTH
            + [pltpu.VMEM((_CH,), jnp.float32),
               pltpu.VMEM((ed,), jnp.float32),
               pltpu.VMEM((_L,), jnp.float32)]
            + [pltpu.SemaphoreType.DMA] * (2 * _DEPTH)
        ),
    )
    b2vec = jnp.full((_L,), b2[0], dtype=jnp.float32)
    out = sc_call(t_t.reshape(2 * n, ed // 2), gl, gr, W2.reshape(ed), b2vec)
    return out.reshape(p, 1)
